# baseline (speedup 1.0000x reference)
import jax
import jax.numpy as jnp
from jax.experimental import pallas as pl
from jax.experimental.pallas import tpu as pltpu

B = 256
N_R = 50000
N_D = 25000
N_S = 12800


def _gat(x, src, dst, n, W, a_s, a_d, act):
    h = x @ W
    es = (h * a_s).sum(-1)
    ed = (h * a_d).sum(-1)
    e = jax.nn.leaky_relu(es[src] + ed[dst], 0.2)
    w = jnp.exp(e)
    den = jax.ops.segment_sum(w, dst, num_segments=n)
    alpha = w / (den[dst] + 1e-9)
    out = jax.ops.segment_sum(alpha[:, None] * h[src], dst, num_segments=n)
    return act(out) + x


def _bn(x, g, b):
    return (x / jnp.sqrt(1.0 + 1e-5)) * g + b


def _refer_mlp_kernel(h_ref, w1_ref, b1_ref, w2_ref, b2_ref, w3_ref, b3_ref, o_ref):
    h = h_ref[...]
    rp = h @ w1_ref[...] + b1_ref[...]
    rp = jnp.maximum(rp, 0.01 * rp)
    rp = rp @ w2_ref[...] + b2_ref[...]
    rp = jnp.maximum(rp, 0.01 * rp)
    o_ref[...] = rp @ w3_ref[...] + b3_ref[...]


def _refer_mlp(h, W1, b1, W2, b2, W3, b3):
    return pl.pallas_call(
        _refer_mlp_kernel,
        out_shape=jax.ShapeDtypeStruct((B, 1), jnp.float32),
    )(h, W1, b1[None, :], W2, b2[None, :], W3, b3[None, :])


def kernel(Smiles_r_node, Smiles_i_node, Smiles_r2r_edge, r2r_edge_index, i2i_edge_index, i2d_src, i2d_dst, d2d_edge_index, Smiles_d2d_edge, r_segment_ids, d_segment_ids, elu1_smiles_r_node, elu1_edge_index, elu1_segment_ids, elu2_smiles_r_node, elu2_edge_index, elu2_segment_ids, W_emb_r, W_emb_i, W_emb_e, W_emb_s, gat_r2r_W, gat_r2r_as, gat_r2r_ad, gat_i2i_W, gat_i2i_as, gat_i2i_ad, gat_solv_W, gat_solv_as, gat_solv_ad, dgcn_W, dgcn_coef, ref_W1, ref_b1, ref_W2, ref_b2, ref_W3, ref_b3, se1_W1, se1_b1, se1_g, se1_be, se1_W2, se1_b2, se2_W1, se2_b1, se2_g, se2_be, se2_W2, se2_b2):
    lrelu = lambda v: jax.nn.leaky_relu(v, 0.01)
    r2 = Smiles_r_node @ W_emb_r
    i_nd = Smiles_r_node @ W_emb_i
    rs, rd = r2r_edge_index[0], r2r_edge_index[1]
    i_s, i_d = i2i_edge_index[0], i2i_edge_index[1]
    for l in range(4):
        r2 = _gat(r2, rs, rd, N_R, gat_r2r_W[l], gat_r2r_as[l], gat_r2r_ad[l], lrelu)
        i_nd = _gat(i_nd, i_s, i_d, N_R, gat_i2i_W[l], gat_i2i_as[l], gat_i2i_ad[l], lrelu)
    d1 = jax.ops.segment_sum(i_nd[i2d_src], i2d_dst, num_segments=N_D)
    hd = d1 @ dgcn_W
    msg = hd[d2d_edge_index[0]] * dgcn_coef[Smiles_d2d_edge][:, None]
    agg = jax.ops.segment_sum(msg, d2d_edge_index[1], num_segments=N_D)
    d2 = lrelu(0.1 * agg + hd)
    h = jax.ops.segment_sum(r2, r_segment_ids, num_segments=B)
    h_exp = h[d_segment_ids]
    refer_p = _refer_mlp(h, ref_W1, ref_b1, ref_W2, ref_b2, ref_W3, ref_b3)
    s1 = elu1_smiles_r_node @ W_emb_s
    e1s, e1d = elu1_edge_index[0], elu1_edge_index[1]
    sh = s1
    for l in range(4):
        sh = _gat(sh, e1s, e1d, N_S, gat_solv_W[l], gat_solv_as[l], gat_solv_ad[l], jax.nn.relu)
    solv_sum = jax.ops.segment_sum(sh, elu1_segment_ids, num_segments=B)
    solv_exp = solv_sum[d_segment_ids]
    v1 = jnp.concatenate([d1, solv_exp], axis=-1)
    t1 = jax.nn.relu(_bn(v1 @ se1_W1 + se1_b1, se1_g, se1_be))
    se1 = jnp.tanh(t1 @ se1_W2 + se1_b2)
    v2 = jnp.concatenate([d2, h_exp, solv_exp], axis=-1)
    t2 = jax.nn.relu(_bn(v2 @ se2_W1 + se2_b1, se2_g, se2_be))
    se2 = jax.nn.sigmoid(t2 @ se2_W2 + se2_b2) * 2.0
    se = se2 * se1
    sum_se = jax.ops.segment_sum(se, d_segment_ids, num_segments=B)
    return refer_p + sum_se
